# Initial kernel scaffold; baseline (speedup 1.0000x reference)
import jax
import jax.numpy as jnp
from jax.experimental import pallas as pl


def kernel(x, edge_index, W1, b1, W2, b2, W3, b3, Wf1, bf1, Wf2, bf2):
    raise NotImplementedError("write your pallas kernel here")


# speedup vs baseline: 12.9037x; 12.9037x over previous
import functools

import jax
import jax.numpy as jnp
from jax import lax
from jax.experimental import pallas as pl
from jax.experimental.pallas import tpu as pltpu
from jax.experimental.pallas import tpu_sc as plsc

NN = 10000
EE = 320000
DIN = 128
HID = 64

NC = 2
NS = 16
NW = NC * NS

NPAD = 10240
CHUNK = 128
NCHUNK = 80
EPT = CHUNK * NCHUNK
EPAD = EPT * NW
RPT = NPAD // NS

_mesh = plsc.VectorSubcoreMesh(core_axis_name="c", subcore_axis_name="s")


def _zero_fill(zero_v, nrow, width):
    z16 = jnp.zeros((16,), jnp.float32)
    for i in range(nrow):
        for k in range(width // 16):
            zero_v[i, pl.ds(k * 16, 16)] = z16


def _zero_acc(acc, zero_v, s, width):
    def zbody(i, _):
        pltpu.sync_copy(zero_v, acc.at[pl.ds(s * RPT + i * 16, 16)])
        return 0
    lax.fori_loop(0, RPT // 16, zbody, 0)


def _copy_out(acc, out_hbm, c, s, buf_v, width):
    def obody(i, _):
        pltpu.sync_copy(acc.at[pl.ds(s * RPT + i * 128, 128)], buf_v)
        pltpu.sync_copy(buf_v, out_hbm.at[c, pl.ds(s * RPT + i * 128, 128)])
        return 0
    lax.fori_loop(0, RPT // 128, obody, 0)


@functools.partial(
    pl.kernel,
    out_type=jax.ShapeDtypeStruct((NC, NPAD, 16), jnp.float32),
    mesh=_mesh,
    scratch_types=[
        pltpu.VMEM((NCHUNK, CHUNK), jnp.int32),
        pltpu.VMEM((CHUNK, 16), jnp.float32),
        pltpu.VMEM((16, 16), jnp.float32),
        pltpu.VMEM((128, 16), jnp.float32),
        pltpu.VMEM_SHARED((NPAD, 16), jnp.float32),
    ],
    compiler_params=pltpu.CompilerParams(use_tc_tiling_on_sc=False),
)
def _deg_kernel(dst_hbm, out_hbm, dst_v, ones_v, zero_v, buf_v, acc):
    c = lax.axis_index("c")
    s = lax.axis_index("s")
    w = c * NS + s
    one_row = jnp.where(lax.iota(jnp.int32, 16) == 0, 1.0, 0.0).astype(jnp.float32)
    for i in range(CHUNK):
        ones_v[i, :] = one_row
    _zero_fill(zero_v, 16, 16)
    pltpu.sync_copy(dst_hbm.at[w], dst_v)
    _zero_acc(acc, zero_v, s, 16)
    plsc.subcore_barrier()

    def body(j, _):
        pltpu.sync_copy(ones_v, acc.at[dst_v.at[j]], add=True)
        return 0
    lax.fori_loop(0, NCHUNK, body, 0)
    plsc.subcore_barrier()
    _copy_out(acc, out_hbm, c, s, buf_v, 16)


@functools.partial(
    pl.kernel,
    out_type=jax.ShapeDtypeStruct((NC, NPAD, HID), jnp.float32),
    mesh=_mesh,
    scratch_types=[
        pltpu.VMEM((NCHUNK, CHUNK), jnp.int32),
        pltpu.VMEM((NCHUNK, CHUNK), jnp.int32),
        pltpu.VMEM((CHUNK, HID), jnp.float32),
        pltpu.VMEM((16, HID), jnp.float32),
        pltpu.VMEM((128, HID), jnp.float32),
        pltpu.VMEM_SHARED((NPAD, HID), jnp.float32),
        pltpu.SemaphoreType.DMA,
    ],
    compiler_params=pltpu.CompilerParams(use_tc_tiling_on_sc=False),
)
def _edge_kernel(g_hbm, src_hbm, dst_hbm, out_hbm,
                 src_v, dst_v, rows_v, zero_v, buf_v, acc, sem):
    c = lax.axis_index("c")
    s = lax.axis_index("s")
    w = c * NS + s
    _zero_fill(zero_v, 16, HID)
    pltpu.sync_copy(src_hbm.at[w], src_v)
    pltpu.sync_copy(dst_hbm.at[w], dst_v)
    _zero_acc(acc, zero_v, s, HID)
    plsc.subcore_barrier()

    def body(j, _):
        pltpu.async_copy(g_hbm.at[src_v.at[j]], rows_v, sem).wait()
        pltpu.sync_copy(rows_v, acc.at[dst_v.at[j]], add=True)
        return 0
    lax.fori_loop(0, NCHUNK, body, 0)
    plsc.subcore_barrier()
    _copy_out(acc, out_hbm, c, s, buf_v, HID)


BLK = 1024
GRID = NPAD // BLK


def _t1_body(deg_ref, x_ref, w1_ref, dinv_ref, g_ref):
    deg = deg_ref[0][:, 0:1] + deg_ref[1][:, 0:1] + 1.0
    dinv = 1.0 / jnp.sqrt(deg)
    g = dinv * jnp.dot(x_ref[...], w1_ref[...], preferred_element_type=jnp.float32)
    dinv_ref[...] = jnp.broadcast_to(dinv, (BLK, HID))
    g_ref[...] = g


def _t2_body(p_ref, g_ref, dinv_ref, b_ref, w_ref, out_ref):
    dinv = dinv_ref[...]
    h = jnp.maximum(dinv * (p_ref[0] + p_ref[1] + g_ref[...]) + b_ref[...], 0.0)
    out_ref[...] = dinv * jnp.dot(h, w_ref[...], preferred_element_type=jnp.float32)


def _t3_body(p_ref, g_ref, dinv_ref, b3_ref, wf1_ref, bf1_ref, wf2_ref, bf2_ref,
             out_ref):
    dinv = dinv_ref[...]
    h3 = jnp.maximum(dinv * (p_ref[0] + p_ref[1] + g_ref[...]) + b3_ref[...], 0.0)
    t = jnp.maximum(jnp.dot(h3, wf1_ref[...], preferred_element_type=jnp.float32)
                    + bf1_ref[...], 0.0)
    logits = (jnp.dot(t, wf2_ref[...], preferred_element_type=jnp.float32)
              + bf2_ref[...])
    m = jnp.max(logits, axis=-1, keepdims=True)
    e = jnp.exp(logits - m)
    out_ref[...] = e / jnp.sum(e, axis=-1, keepdims=True)


def _row_spec(width):
    return pl.BlockSpec((BLK, width), lambda i: (i, 0))


def _part_spec(width):
    return pl.BlockSpec((NC, BLK, width), lambda i: (0, i, 0))


def _full_spec(a, b):
    return pl.BlockSpec((a, b), lambda i: (0, 0))


def kernel(x, edge_index, W1, b1, W2, b2, W3, b3, Wf1, bf1, Wf2, bf2):
    f32 = jnp.float32
    src = edge_index[0]
    dst = edge_index[1]
    npad_e = EPAD - EE
    src_p = jnp.concatenate([src, jnp.zeros((npad_e,), jnp.int32)])
    dst_p = jnp.concatenate([dst, jnp.full((npad_e,), NN, jnp.int32)])
    src_t = src_p.reshape(NW, NCHUNK, CHUNK)
    dst_t = dst_p.reshape(NW, NCHUNK, CHUNK)

    x_p = jnp.zeros((NPAD, DIN), f32).at[:NN].set(x)

    deg_part = _deg_kernel(dst_t)

    dinv, g1 = pl.pallas_call(
        _t1_body,
        grid=(GRID,),
        in_specs=[_part_spec(16), _row_spec(DIN), _full_spec(DIN, HID)],
        out_specs=[_row_spec(HID), _row_spec(HID)],
        out_shape=[jax.ShapeDtypeStruct((NPAD, HID), f32),
                   jax.ShapeDtypeStruct((NPAD, HID), f32)],
    )(deg_part, x_p, W1)

    def combine(part, g, b, w):
        return pl.pallas_call(
            _t2_body,
            grid=(GRID,),
            in_specs=[_part_spec(HID), _row_spec(HID), _row_spec(HID),
                      _full_spec(1, HID), _full_spec(HID, HID)],
            out_specs=_row_spec(HID),
            out_shape=jax.ShapeDtypeStruct((NPAD, HID), f32),
        )(part, g, dinv, b.reshape(1, HID), w)

    s1 = _edge_kernel(g1, src_t, dst_t)
    g2 = combine(s1, g1, b1, W2)
    s2 = _edge_kernel(g2, src_t, dst_t)
    g3 = combine(s2, g2, b2, W3)
    s3 = _edge_kernel(g3, src_t, dst_t)

    wf2_p = jnp.zeros((HID, 128), f32).at[:, :3].set(Wf2)
    bf2_p = jnp.full((1, 128), -1e30, f32).at[0, :3].set(bf2)
    probs = pl.pallas_call(
        _t3_body,
        grid=(GRID,),
        in_specs=[_part_spec(HID), _row_spec(HID), _row_spec(HID),
                  _full_spec(1, HID), _full_spec(HID, HID), _full_spec(1, HID),
                  _full_spec(HID, 128), _full_spec(1, 128)],
        out_specs=_row_spec(128),
        out_shape=jax.ShapeDtypeStruct((NPAD, 128), f32),
    )(s3, g3, dinv, b3.reshape(1, HID), Wf1, bf1.reshape(1, HID), wf2_p, bf2_p)
    return probs[:NN, :3]


# speedup vs baseline: 15.3342x; 1.1884x over previous
import functools

import jax
import jax.numpy as jnp
from jax import lax
from jax.experimental import pallas as pl
from jax.experimental.pallas import tpu as pltpu
from jax.experimental.pallas import tpu_sc as plsc

NN = 10000
EE = 320000
DIN = 128
HID = 64

NC = 2
NS = 16
NW = NC * NS

NPAD = 10240
CHUNK = 128
NCHUNK = 80
EPT = CHUNK * NCHUNK
EPAD = EPT * NW
RPT = NPAD // NS

_mesh = plsc.VectorSubcoreMesh(core_axis_name="c", subcore_axis_name="s")


def _zero_fill(zero_v, nrow, width):
    z16 = jnp.zeros((16,), jnp.float32)
    for i in range(nrow):
        for k in range(width // 16):
            zero_v[i, pl.ds(k * 16, 16)] = z16


def _zero_acc(acc, zero_v, s, width):
    def zbody(i, _):
        pltpu.sync_copy(zero_v, acc.at[pl.ds(s * RPT + i * 16, 16)])
        return 0
    lax.fori_loop(0, RPT // 16, zbody, 0)


def _copy_out(acc, out_hbm, c, s, buf_v, width):
    def obody(i, _):
        pltpu.sync_copy(acc.at[pl.ds(s * RPT + i * 128, 128)], buf_v)
        pltpu.sync_copy(buf_v, out_hbm.at[c, pl.ds(s * RPT + i * 128, 128)])
        return 0
    lax.fori_loop(0, RPT // 128, obody, 0)


@functools.partial(
    pl.kernel,
    out_type=jax.ShapeDtypeStruct((NC, NPAD, 16), jnp.float32),
    mesh=_mesh,
    scratch_types=[
        pltpu.VMEM((NCHUNK, CHUNK), jnp.int32),
        pltpu.VMEM((CHUNK, 16), jnp.float32),
        pltpu.VMEM((16, 16), jnp.float32),
        pltpu.VMEM((128, 16), jnp.float32),
        pltpu.VMEM_SHARED((NPAD, 16), jnp.float32),
    ],
    compiler_params=pltpu.CompilerParams(use_tc_tiling_on_sc=False),
)
def _deg_kernel(dst_hbm, out_hbm, dst_v, ones_v, zero_v, buf_v, acc):
    c = lax.axis_index("c")
    s = lax.axis_index("s")
    w = c * NS + s
    one_row = jnp.where(lax.iota(jnp.int32, 16) == 0, 1.0, 0.0).astype(jnp.float32)
    for i in range(CHUNK):
        ones_v[i, :] = one_row
    _zero_fill(zero_v, 16, 16)
    pltpu.sync_copy(dst_hbm.at[w], dst_v)
    _zero_acc(acc, zero_v, s, 16)
    plsc.subcore_barrier()

    def body(j, _):
        pltpu.sync_copy(ones_v, acc.at[dst_v.at[j]], add=True)
        return 0
    lax.fori_loop(0, NCHUNK, body, 0)
    plsc.subcore_barrier()
    _copy_out(acc, out_hbm, c, s, buf_v, 16)


NBUF = 4
GROUPS = NCHUNK // NBUF


@functools.partial(
    pl.kernel,
    out_type=jax.ShapeDtypeStruct((NC, NPAD, HID), jnp.float32),
    mesh=_mesh,
    scratch_types=[
        pltpu.VMEM((NCHUNK, CHUNK), jnp.int32),
        pltpu.VMEM((NCHUNK, CHUNK), jnp.int32),
        pltpu.VMEM((NBUF, CHUNK, HID), jnp.float32),
        pltpu.VMEM((16, HID), jnp.float32),
        pltpu.VMEM((128, HID), jnp.float32),
        pltpu.VMEM_SHARED((NPAD, HID), jnp.float32),
        pltpu.SemaphoreType.DMA,
        pltpu.SemaphoreType.DMA,
        pltpu.SemaphoreType.DMA,
        pltpu.SemaphoreType.DMA,
        pltpu.SemaphoreType.DMA,
        pltpu.SemaphoreType.DMA,
        pltpu.SemaphoreType.DMA,
        pltpu.SemaphoreType.DMA,
    ],
    compiler_params=pltpu.CompilerParams(use_tc_tiling_on_sc=False),
)
def _edge_kernel(g_hbm, src_hbm, dst_hbm, out_hbm,
                 src_v, dst_v, rows_v, zero_v, buf_v, acc,
                 gs0, gs1, gs2, gs3, ss0, ss1, ss2, ss3):
    c = lax.axis_index("c")
    s = lax.axis_index("s")
    w = c * NS + s
    gsems = (gs0, gs1, gs2, gs3)
    ssems = (ss0, ss1, ss2, ss3)
    _zero_fill(zero_v, 16, HID)
    pltpu.sync_copy(src_hbm.at[w], src_v)
    pltpu.sync_copy(dst_hbm.at[w], dst_v)
    _zero_acc(acc, zero_v, s, HID)
    plsc.subcore_barrier()

    for b in range(NBUF):
        pltpu.async_copy(g_hbm.at[src_v.at[b]], rows_v.at[b], gsems[b])

    def body(i, _):
        for b in range(NBUF):
            j = i * NBUF + b
            pltpu.make_async_copy(g_hbm.at[src_v.at[j]], rows_v.at[b],
                                  gsems[b]).wait()
            pltpu.async_copy(rows_v.at[b], acc.at[dst_v.at[j]], ssems[b],
                             add=True)

            @pl.when(i < GROUPS - 1)
            def _():
                pltpu.make_async_copy(rows_v.at[b], acc.at[dst_v.at[j]],
                                      ssems[b]).wait()
                pltpu.async_copy(g_hbm.at[src_v.at[j + NBUF]], rows_v.at[b],
                                 gsems[b])
        return 0
    lax.fori_loop(0, GROUPS, body, 0)
    for b in range(NBUF):
        pltpu.make_async_copy(rows_v.at[b],
                              acc.at[dst_v.at[NCHUNK - NBUF + b]],
                              ssems[b]).wait()
    plsc.subcore_barrier()
    _copy_out(acc, out_hbm, c, s, buf_v, HID)


BLK = 1024
GRID = NPAD // BLK


def _t1_body(deg_ref, x_ref, w1_ref, dinv_ref, g_ref):
    deg = deg_ref[0][:, 0:1] + deg_ref[1][:, 0:1] + 1.0
    dinv = 1.0 / jnp.sqrt(deg)
    g = dinv * jnp.dot(x_ref[...], w1_ref[...], preferred_element_type=jnp.float32)
    dinv_ref[...] = jnp.broadcast_to(dinv, (BLK, HID))
    g_ref[...] = g


def _t2_body(p_ref, g_ref, dinv_ref, b_ref, w_ref, out_ref):
    dinv = dinv_ref[...]
    h = jnp.maximum(dinv * (p_ref[0] + p_ref[1] + g_ref[...]) + b_ref[...], 0.0)
    out_ref[...] = dinv * jnp.dot(h, w_ref[...], preferred_element_type=jnp.float32)


def _t3_body(p_ref, g_ref, dinv_ref, b3_ref, wf1_ref, bf1_ref, wf2_ref, bf2_ref,
             out_ref):
    dinv = dinv_ref[...]
    h3 = jnp.maximum(dinv * (p_ref[0] + p_ref[1] + g_ref[...]) + b3_ref[...], 0.0)
    t = jnp.maximum(jnp.dot(h3, wf1_ref[...], preferred_element_type=jnp.float32)
                    + bf1_ref[...], 0.0)
    logits = (jnp.dot(t, wf2_ref[...], preferred_element_type=jnp.float32)
              + bf2_ref[...])
    m = jnp.max(logits, axis=-1, keepdims=True)
    e = jnp.exp(logits - m)
    out_ref[...] = e / jnp.sum(e, axis=-1, keepdims=True)


def _row_spec(width):
    return pl.BlockSpec((BLK, width), lambda i: (i, 0))


def _part_spec(width):
    return pl.BlockSpec((NC, BLK, width), lambda i: (0, i, 0))


def _full_spec(a, b):
    return pl.BlockSpec((a, b), lambda i: (0, 0))


def kernel(x, edge_index, W1, b1, W2, b2, W3, b3, Wf1, bf1, Wf2, bf2):
    f32 = jnp.float32
    src = edge_index[0]
    dst = edge_index[1]
    npad_e = EPAD - EE
    src_p = jnp.concatenate([src, jnp.zeros((npad_e,), jnp.int32)])
    dst_p = jnp.concatenate([dst, jnp.full((npad_e,), NN, jnp.int32)])
    src_t = src_p.reshape(NW, NCHUNK, CHUNK)
    dst_t = dst_p.reshape(NW, NCHUNK, CHUNK)

    x_p = jnp.zeros((NPAD, DIN), f32).at[:NN].set(x)

    deg_part = _deg_kernel(dst_t)

    dinv, g1 = pl.pallas_call(
        _t1_body,
        grid=(GRID,),
        in_specs=[_part_spec(16), _row_spec(DIN), _full_spec(DIN, HID)],
        out_specs=[_row_spec(HID), _row_spec(HID)],
        out_shape=[jax.ShapeDtypeStruct((NPAD, HID), f32),
                   jax.ShapeDtypeStruct((NPAD, HID), f32)],
    )(deg_part, x_p, W1)

    def combine(part, g, b, w):
        return pl.pallas_call(
            _t2_body,
            grid=(GRID,),
            in_specs=[_part_spec(HID), _row_spec(HID), _row_spec(HID),
                      _full_spec(1, HID), _full_spec(HID, HID)],
            out_specs=_row_spec(HID),
            out_shape=jax.ShapeDtypeStruct((NPAD, HID), f32),
        )(part, g, dinv, b.reshape(1, HID), w)

    s1 = _edge_kernel(g1, src_t, dst_t)
    g2 = combine(s1, g1, b1, W2)
    s2 = _edge_kernel(g2, src_t, dst_t)
    g3 = combine(s2, g2, b2, W3)
    s3 = _edge_kernel(g3, src_t, dst_t)

    wf2_p = jnp.zeros((HID, 128), f32).at[:, :3].set(Wf2)
    bf2_p = jnp.full((1, 128), -1e30, f32).at[0, :3].set(bf2)
    probs = pl.pallas_call(
        _t3_body,
        grid=(GRID,),
        in_specs=[_part_spec(HID), _row_spec(HID), _row_spec(HID),
                  _full_spec(1, HID), _full_spec(HID, HID), _full_spec(1, HID),
                  _full_spec(HID, 128), _full_spec(1, 128)],
        out_specs=_row_spec(128),
        out_shape=jax.ShapeDtypeStruct((NPAD, 128), f32),
    )(s3, g3, dinv, b3.reshape(1, HID), Wf1, bf1.reshape(1, HID), wf2_p, bf2_p)
    return probs[:NN, :3]
